# baseline (speedup 1.0000x reference)
import functools

import jax
import jax.numpy as jnp
from jax import lax
from jax.experimental import pallas as pl
from jax.experimental.pallas import tpu as pltpu
from jax.experimental.pallas import tpu_sc as plsc

H = 2048
E = 8
TOPK = 2
MOE_I = 1408
SH_I = 5632
T = 2048


_RT = 256


def _router_body(x_ref, gw_ref, sel_ref, w_ref, xb_ref):
    x = x_ref[...]
    logits = jnp.dot(x, gw_ref[...], preferred_element_type=jnp.float32)
    m = jnp.max(logits, axis=1, keepdims=True)
    p = jnp.exp(logits - m)
    p = p / jnp.sum(p, axis=1, keepdims=True)
    col = lax.broadcasted_iota(jnp.int32, p.shape, 1)
    m1 = jnp.max(p, axis=1, keepdims=True)
    i1 = jnp.min(jnp.where(p == m1, col, E), axis=1, keepdims=True)
    pm = jnp.where(col == i1, -1.0, p)
    m2 = jnp.max(pm, axis=1, keepdims=True)
    i2 = jnp.min(jnp.where(pm == m2, col, E), axis=1, keepdims=True)
    sel_ref[...] = jnp.where(col == 0, i1, jnp.where(col == 1, i2, 0))
    w_ref[...] = jnp.where(col == 0, m1, jnp.where(col == 1, m2, 0.0))
    xb_ref[...] = x.astype(jnp.bfloat16)


def _router(x, gate_w):
    return pl.pallas_call(
        _router_body,
        grid=(T // _RT,),
        in_specs=[
            pl.BlockSpec((_RT, H), lambda t: (t, 0)),
            pl.BlockSpec((H, E), lambda t: (0, 0)),
        ],
        out_specs=[
            pl.BlockSpec((_RT, E), lambda t: (t, 0)),
            pl.BlockSpec((_RT, E), lambda t: (t, 0)),
            pl.BlockSpec((_RT, H), lambda t: (t, 0)),
        ],
        out_shape=[
            jax.ShapeDtypeStruct((T, E), jnp.int32),
            jax.ShapeDtypeStruct((T, E), jnp.float32),
            jax.ShapeDtypeStruct((T, H), jnp.bfloat16),
        ],
    )(x, gate_w)


_TM2 = 256
_NT2 = T // _TM2
_IC2 = 512
_K2 = SH_I // _IC2


def _shared_body(x_ref, g_ref, u_ref, d_ref, sg_ref, out_ref, acc_ref):
    k = pl.program_id(0)
    t = pl.program_id(1)

    x = x_ref[...]
    g = jnp.dot(x, g_ref[...], preferred_element_type=jnp.float32)
    u = jnp.dot(x, u_ref[...], preferred_element_type=jnp.float32)
    a = (g * jax.nn.sigmoid(g) * u).astype(jnp.bfloat16)
    part = jnp.dot(a, d_ref[...], preferred_element_type=jnp.float32)
    sl = pl.ds(t * _TM2, _TM2)

    @pl.when(k == 0)
    def _():
        acc_ref[sl, :] = part

    @pl.when(k > 0)
    def _():
        acc_ref[sl, :] += part

    @pl.when(k == _K2 - 1)
    def _():
        sw = jax.nn.sigmoid(
            jnp.dot(x, sg_ref[...], preferred_element_type=jnp.float32))
        out_ref[...] = acc_ref[sl, :] * sw


def _shared(xb, shared_gu_b, shared_down_b, shared_gate_b):
    return pl.pallas_call(
        _shared_body,
        grid=(_K2, _NT2),
        in_specs=[
            pl.BlockSpec((_TM2, H), lambda k, t: (t, 0)),
            pl.BlockSpec((H, _IC2), lambda k, t: (0, k)),
            pl.BlockSpec((H, _IC2), lambda k, t: (0, _K2 + k)),
            pl.BlockSpec((_IC2, H), lambda k, t: (k, 0)),
            pl.BlockSpec((H, 1), lambda k, t: (0, 0)),
        ],
        out_specs=pl.BlockSpec((_TM2, H), lambda k, t: (t, 0)),
        out_shape=jax.ShapeDtypeStruct((T, H), jnp.float32),
        scratch_shapes=[pltpu.VMEM((T, H), jnp.float32)],
    )(xb, shared_gu_b, shared_gu_b, shared_down_b, shared_gate_b)


_TM = 128
_NT = T * TOPK // _TM + E - 1
_PAD = _NT * _TM
_PAD_SC = ((_PAD + 255) // 256) * 256


def _group_body(eot_ref, x_ref, g_ref, u_ref, d_ref, w_ref, out_ref):
    x = x_ref[...].astype(jnp.bfloat16)
    g = jnp.dot(x, g_ref[0], preferred_element_type=jnp.float32)
    u = jnp.dot(x, u_ref[0], preferred_element_type=jnp.float32)
    a = (g * jax.nn.sigmoid(g) * u).astype(jnp.bfloat16)
    y = jnp.dot(a, d_ref[0], preferred_element_type=jnp.float32)
    out_ref[...] = y * w_ref[...]


def _grouped(xs, experts_gu_b, experts_down_b, w_pad, eot):
    grid_spec = pltpu.PrefetchScalarGridSpec(
        num_scalar_prefetch=1,
        grid=(_NT,),
        in_specs=[
            pl.BlockSpec((_TM, H), lambda t, eot: (t, 0)),
            pl.BlockSpec((1, H, MOE_I), lambda t, eot: (eot[t], 0, 0)),
            pl.BlockSpec((1, H, MOE_I), lambda t, eot: (eot[t], 0, 1)),
            pl.BlockSpec((1, MOE_I, H), lambda t, eot: (eot[t], 0, 0)),
            pl.BlockSpec((_TM, 1), lambda t, eot: (t, 0)),
        ],
        out_specs=pl.BlockSpec((_TM, H), lambda t, eot: (t, 0)),
    )
    return pl.pallas_call(
        _group_body,
        grid_spec=grid_spec,
        out_shape=jax.ShapeDtypeStruct((_PAD, H), jnp.float32),
    )(eot, xs, experts_gu_b, experts_gu_b, experts_down_b, w_pad)


_CH = 16


def _make_sc_gather(B, dtype):
    info = plsc.get_sparse_core_info()
    nc, ns = info.num_cores, info.num_subcores
    nw = nc * ns
    b_per_w = B // nw
    n_chunks = b_per_w // _CH
    mesh = plsc.VectorSubcoreMesh(core_axis_name="c", subcore_axis_name="s")

    @functools.partial(
        pl.kernel,
        mesh=mesh,
        out_type=jax.ShapeDtypeStruct((B, H), dtype),
        scratch_types=[
            pltpu.VMEM((b_per_w,), jnp.int32),
            pltpu.VMEM((_CH, H), dtype),
            pltpu.VMEM((_CH, H), dtype),
            pltpu.SemaphoreType.DMA,
            pltpu.SemaphoreType.DMA,
            pltpu.SemaphoreType.DMA,
            pltpu.SemaphoreType.DMA,
        ],
    )
    def k(table_hbm, idx_hbm, out_hbm, idx_v, rv0, rv1, g0, g1, s0, s1):
        wid = lax.axis_index("s") * nc + lax.axis_index("c")
        base = wid * b_per_w
        pltpu.sync_copy(idx_hbm.at[pl.ds(base, b_per_w)], idx_v)
        rv = (rv0, rv1)
        gsem = (g0, g1)
        ssem = (s0, s1)
        gh = [None, None]
        sh = [None, None]
        for c in range(n_chunks):
            b = c % 2
            if c >= 2:
                sh[b].wait()
            gh[b] = pltpu.async_copy(
                table_hbm.at[idx_v.at[pl.ds(c * _CH, _CH)]], rv[b], gsem[b])
            if c >= 1:
                pb = (c - 1) % 2
                gh[pb].wait()
                sh[pb] = pltpu.async_copy(
                    rv[pb], out_hbm.at[pl.ds(base + (c - 1) * _CH, _CH)],
                    ssem[pb])
        lb = (n_chunks - 1) % 2
        gh[lb].wait()
        sh[lb] = pltpu.async_copy(
            rv[lb], out_hbm.at[pl.ds(base + (n_chunks - 1) * _CH, _CH)],
            ssem[lb])
        if n_chunks >= 2:
            sh[(n_chunks - 2) % 2].wait()
        sh[lb].wait()

    return k


_TA = 128


def _add_body(sh_ref, g0_ref, g1_ref, out_ref):
    out_ref[...] = sh_ref[...] + g0_ref[...] + g1_ref[...]


def _combine(sh, gcat):
    return pl.pallas_call(
        _add_body,
        grid=(T // _TA,),
        in_specs=[
            pl.BlockSpec((_TA, H), lambda t: (t, 0)),
            pl.BlockSpec((_TA, H), lambda t: (t, 0)),
            pl.BlockSpec((_TA, H), lambda t: (t + T // _TA, 0)),
        ],
        out_specs=pl.BlockSpec((_TA, H), lambda t: (t, 0)),
        out_shape=jax.ShapeDtypeStruct((T, H), jnp.float32),
    )(sh, gcat, gcat)


def kernel(hidden_states, gate_w, shared_gate_w, shared_gu_w, shared_down_w,
           experts_gu_w, experts_down_w):
    orig_shape = hidden_states.shape
    x = hidden_states.reshape(T, H)

    sel8, w8, xb = _router(x, gate_w)
    sel = sel8[:, :TOPK]
    wts = w8[:, :TOPK]

    a = T * TOPK
    e_flat = sel.reshape(a)
    w_flat = wts.reshape(a)
    perm = jnp.argsort(e_flat, stable=True)
    e_sorted = e_flat[perm]
    counts = jnp.zeros((E,), jnp.int32).at[e_flat].add(1)
    padded = ((counts + _TM - 1) // _TM) * _TM
    pad_ends = jnp.cumsum(padded)
    p_start = pad_ends - padded
    c_start = jnp.cumsum(counts) - counts
    rank = jnp.arange(a, dtype=jnp.int32) - c_start[e_sorted]
    pos_sorted = p_start[e_sorted] + rank
    pos = jnp.zeros((a,), jnp.int32).at[perm].set(pos_sorted)
    src_tok = jnp.zeros((_PAD_SC,), jnp.int32).at[pos_sorted].set(
        (perm // TOPK).astype(jnp.int32))
    w_pad = jnp.zeros((_PAD, 1), jnp.float32).at[pos_sorted, 0].set(
        w_flat[perm])
    eot = jnp.minimum(
        jnp.searchsorted(pad_ends, jnp.arange(_NT, dtype=jnp.int32) * _TM,
                         side='right'),
        E - 1).astype(jnp.int32)
    pos2 = pos.reshape(T, TOPK)
    g_idx = jnp.concatenate([pos2[:, 0], pos2[:, 1]])

    experts_gu_b = experts_gu_w.astype(jnp.bfloat16)
    experts_down_b = experts_down_w.astype(jnp.bfloat16)
    shared_gu_b = shared_gu_w.astype(jnp.bfloat16)
    shared_down_b = shared_down_w.astype(jnp.bfloat16)
    shared_gate_b = shared_gate_w.astype(jnp.bfloat16)

    xs = _make_sc_gather(_PAD_SC, jnp.float32)(x, src_tok)

    ys = _grouped(xs, experts_gu_b, experts_down_b, w_pad, eot)

    gcat = _make_sc_gather(T * TOPK, jnp.float32)(ys, g_idx)

    sh = _shared(xb, shared_gu_b, shared_down_b, shared_gate_b)
    out = _combine(sh, gcat)
    return out.reshape(orig_shape)


# speedup vs baseline: 1.1885x; 1.1885x over previous
import functools

import jax
import jax.numpy as jnp
from jax import lax
from jax.experimental import pallas as pl
from jax.experimental.pallas import tpu as pltpu
from jax.experimental.pallas import tpu_sc as plsc

H = 2048
E = 8
TOPK = 2
MOE_I = 1408
SH_I = 5632
T = 2048


_RT = 256


def _router_body(x_ref, gw_ref, sel_ref, w_ref):
    x = x_ref[...]
    logits = jnp.dot(x, gw_ref[...], preferred_element_type=jnp.float32)
    m = jnp.max(logits, axis=1, keepdims=True)
    p = jnp.exp(logits - m)
    p = p / jnp.sum(p, axis=1, keepdims=True)
    col = lax.broadcasted_iota(jnp.int32, p.shape, 1)
    m1 = jnp.max(p, axis=1, keepdims=True)
    i1 = jnp.min(jnp.where(p == m1, col, E), axis=1, keepdims=True)
    pm = jnp.where(col == i1, -1.0, p)
    m2 = jnp.max(pm, axis=1, keepdims=True)
    i2 = jnp.min(jnp.where(pm == m2, col, E), axis=1, keepdims=True)
    sel_ref[...] = jnp.where(col == 0, i1, jnp.where(col == 1, i2, 0))
    w_ref[...] = jnp.where(col == 0, m1, jnp.where(col == 1, m2, 0.0))


def _router(x, gate_w):
    return pl.pallas_call(
        _router_body,
        grid=(T // _RT,),
        in_specs=[
            pl.BlockSpec((_RT, H), lambda t: (t, 0)),
            pl.BlockSpec((H, E), lambda t: (0, 0)),
        ],
        out_specs=[
            pl.BlockSpec((_RT, E), lambda t: (t, 0)),
            pl.BlockSpec((_RT, E), lambda t: (t, 0)),
        ],
        out_shape=[
            jax.ShapeDtypeStruct((T, E), jnp.int32),
            jax.ShapeDtypeStruct((T, E), jnp.float32),
        ],
    )(x, gate_w)


_TM2 = 256
_NT2 = T // _TM2
_IC2 = 512
_K2 = SH_I // _IC2


def _shared_body(x_ref, g_ref, u_ref, d_ref, sg_ref, out_ref, acc_ref):
    k = pl.program_id(0)
    t = pl.program_id(1)

    x = x_ref[...]
    g = jnp.dot(x, g_ref[...], preferred_element_type=jnp.float32)
    u = jnp.dot(x, u_ref[...], preferred_element_type=jnp.float32)
    a = g * jax.nn.sigmoid(g) * u
    part = jnp.dot(a, d_ref[...], preferred_element_type=jnp.float32)
    sl = pl.ds(t * _TM2, _TM2)

    @pl.when(k == 0)
    def _():
        acc_ref[sl, :] = part

    @pl.when(k > 0)
    def _():
        acc_ref[sl, :] += part

    @pl.when(k == _K2 - 1)
    def _():
        sw = jax.nn.sigmoid(
            jnp.dot(x, sg_ref[...], preferred_element_type=jnp.float32))
        out_ref[...] = acc_ref[sl, :] * sw


def _shared(x, shared_gu_w, shared_down_w, shared_gate_w):
    return pl.pallas_call(
        _shared_body,
        grid=(_K2, _NT2),
        in_specs=[
            pl.BlockSpec((_TM2, H), lambda k, t: (t, 0)),
            pl.BlockSpec((H, _IC2), lambda k, t: (0, k)),
            pl.BlockSpec((H, _IC2), lambda k, t: (0, _K2 + k)),
            pl.BlockSpec((_IC2, H), lambda k, t: (k, 0)),
            pl.BlockSpec((H, 1), lambda k, t: (0, 0)),
        ],
        out_specs=pl.BlockSpec((_TM2, H), lambda k, t: (t, 0)),
        out_shape=jax.ShapeDtypeStruct((T, H), jnp.float32),
        scratch_shapes=[pltpu.VMEM((T, H), jnp.float32)],
    )(x, shared_gu_w, shared_gu_w, shared_down_w, shared_gate_w)


_TM = 128
_NT = T * TOPK // _TM + E - 1
_PAD = _NT * _TM
_PAD_SC = ((_PAD + 255) // 256) * 256


def _act_body(eot_ref, x_ref, g_ref, u_ref, out_ref):
    x = x_ref[...]
    g = jnp.dot(x, g_ref[0], preferred_element_type=jnp.float32)
    u = jnp.dot(x, u_ref[0], preferred_element_type=jnp.float32)
    out_ref[...] = g * jax.nn.sigmoid(g) * u


def _down_body(eot_ref, a_ref, d_ref, w_ref, out_ref):
    y = jnp.dot(a_ref[...], d_ref[0], preferred_element_type=jnp.float32)
    out_ref[...] = y * w_ref[...]


def _grouped(xs, experts_gu_w, experts_down_w, w_pad, eot):
    act_spec = pltpu.PrefetchScalarGridSpec(
        num_scalar_prefetch=1,
        grid=(_NT,),
        in_specs=[
            pl.BlockSpec((_TM, H), lambda t, eot: (t, 0)),
            pl.BlockSpec((1, H, MOE_I), lambda t, eot: (eot[t], 0, 0)),
            pl.BlockSpec((1, H, MOE_I), lambda t, eot: (eot[t], 0, 1)),
        ],
        out_specs=pl.BlockSpec((_TM, MOE_I), lambda t, eot: (t, 0)),
    )
    act = pl.pallas_call(
        _act_body,
        grid_spec=act_spec,
        out_shape=jax.ShapeDtypeStruct((_PAD, MOE_I), jnp.float32),
    )(eot, xs, experts_gu_w, experts_gu_w)
    down_spec = pltpu.PrefetchScalarGridSpec(
        num_scalar_prefetch=1,
        grid=(_NT,),
        in_specs=[
            pl.BlockSpec((_TM, MOE_I), lambda t, eot: (t, 0)),
            pl.BlockSpec((1, MOE_I, H), lambda t, eot: (eot[t], 0, 0)),
            pl.BlockSpec((_TM, 1), lambda t, eot: (t, 0)),
        ],
        out_specs=pl.BlockSpec((_TM, H), lambda t, eot: (t, 0)),
    )
    return pl.pallas_call(
        _down_body,
        grid_spec=down_spec,
        out_shape=jax.ShapeDtypeStruct((_PAD, H), jnp.float32),
    )(eot, act, experts_down_w, w_pad)


_CH = 16


def _make_sc_gather(B, dtype):
    info = plsc.get_sparse_core_info()
    nc, ns = info.num_cores, info.num_subcores
    nw = nc * ns
    b_per_w = B // nw
    n_chunks = b_per_w // _CH
    mesh = plsc.VectorSubcoreMesh(core_axis_name="c", subcore_axis_name="s")

    @functools.partial(
        pl.kernel,
        mesh=mesh,
        out_type=jax.ShapeDtypeStruct((B, H), dtype),
        scratch_types=[
            pltpu.VMEM((b_per_w,), jnp.int32),
            pltpu.VMEM((_CH, H), dtype),
            pltpu.VMEM((_CH, H), dtype),
            pltpu.SemaphoreType.DMA,
            pltpu.SemaphoreType.DMA,
            pltpu.SemaphoreType.DMA,
            pltpu.SemaphoreType.DMA,
        ],
    )
    def k(table_hbm, idx_hbm, out_hbm, idx_v, rv0, rv1, g0, g1, s0, s1):
        wid = lax.axis_index("s") * nc + lax.axis_index("c")
        base = wid * b_per_w
        pltpu.sync_copy(idx_hbm.at[pl.ds(base, b_per_w)], idx_v)
        rv = (rv0, rv1)
        gsem = (g0, g1)
        ssem = (s0, s1)
        gh = [None, None]
        sh = [None, None]
        for c in range(n_chunks):
            b = c % 2
            if c >= 2:
                sh[b].wait()
            gh[b] = pltpu.async_copy(
                table_hbm.at[idx_v.at[pl.ds(c * _CH, _CH)]], rv[b], gsem[b])
            if c >= 1:
                pb = (c - 1) % 2
                gh[pb].wait()
                sh[pb] = pltpu.async_copy(
                    rv[pb], out_hbm.at[pl.ds(base + (c - 1) * _CH, _CH)],
                    ssem[pb])
        lb = (n_chunks - 1) % 2
        gh[lb].wait()
        sh[lb] = pltpu.async_copy(
            rv[lb], out_hbm.at[pl.ds(base + (n_chunks - 1) * _CH, _CH)],
            ssem[lb])
        if n_chunks >= 2:
            sh[(n_chunks - 2) % 2].wait()
        sh[lb].wait()

    return k


_TA = 128


def _add_body(sh_ref, g0_ref, g1_ref, out_ref):
    out_ref[...] = sh_ref[...] + g0_ref[...] + g1_ref[...]


def _combine(sh, gcat):
    return pl.pallas_call(
        _add_body,
        grid=(T // _TA,),
        in_specs=[
            pl.BlockSpec((_TA, H), lambda t: (t, 0)),
            pl.BlockSpec((_TA, H), lambda t: (t, 0)),
            pl.BlockSpec((_TA, H), lambda t: (t + T // _TA, 0)),
        ],
        out_specs=pl.BlockSpec((_TA, H), lambda t: (t, 0)),
        out_shape=jax.ShapeDtypeStruct((T, H), jnp.float32),
    )(sh, gcat, gcat)


def kernel(hidden_states, gate_w, shared_gate_w, shared_gu_w, shared_down_w,
           experts_gu_w, experts_down_w):
    orig_shape = hidden_states.shape
    x = hidden_states.reshape(T, H)

    sel8, w8 = _router(x, gate_w)
    sel = sel8[:, :TOPK]
    wts = w8[:, :TOPK]

    a = T * TOPK
    e_flat = sel.reshape(a)
    w_flat = wts.reshape(a)
    perm = jnp.argsort(e_flat, stable=True)
    e_sorted = e_flat[perm]
    counts = jnp.zeros((E,), jnp.int32).at[e_flat].add(1)
    padded = ((counts + _TM - 1) // _TM) * _TM
    pad_ends = jnp.cumsum(padded)
    p_start = pad_ends - padded
    c_start = jnp.cumsum(counts) - counts
    rank = jnp.arange(a, dtype=jnp.int32) - c_start[e_sorted]
    pos_sorted = p_start[e_sorted] + rank
    pos = jnp.zeros((a,), jnp.int32).at[perm].set(pos_sorted)
    src_tok = jnp.zeros((_PAD_SC,), jnp.int32).at[pos_sorted].set(
        (perm // TOPK).astype(jnp.int32))
    w_pad = jnp.zeros((_PAD, 1), jnp.float32).at[pos_sorted, 0].set(
        w_flat[perm])
    eot = jnp.minimum(
        jnp.searchsorted(pad_ends, jnp.arange(_NT, dtype=jnp.int32) * _TM,
                         side='right'),
        E - 1).astype(jnp.int32)
    pos2 = pos.reshape(T, TOPK)
    g_idx = jnp.concatenate([pos2[:, 0], pos2[:, 1]])

    xs = _make_sc_gather(_PAD_SC, jnp.float32)(x, src_tok)

    ys = _grouped(xs, experts_gu_w, experts_down_w, w_pad, eot)

    gcat = _make_sc_gather(T * TOPK, jnp.float32)(ys, g_idx)

    sh = _shared(x, shared_gu_w, shared_down_w, shared_gate_w)
    out = _combine(sh, gcat)
    return out.reshape(orig_shape)
